# baseline (speedup 1.0000x reference)
import functools

import jax
import jax.numpy as jnp
from jax import lax
from jax.experimental import pallas as pl
from jax.experimental.pallas import tpu as pltpu
from jax.experimental.pallas import tpu_sc as plsc

NC = 2
NS = 16
L = 16

CHUNK = 128


def _sc_aggregate(n_rows_pad, n_chunk_rows, d_half):
    rows_per_tile = n_rows_pad // NS
    chunks_per_tile = n_chunk_rows // NS
    n_half = chunks_per_tile // 2
    mesh = plsc.VectorSubcoreMesh(
        core_axis_name="c", subcore_axis_name="s", num_cores=NC, num_subcores=NS)

    @functools.partial(
        pl.kernel,
        out_type=(
            jax.ShapeDtypeStruct((n_rows_pad, d_half), jnp.float32),
            jax.ShapeDtypeStruct((n_rows_pad, d_half), jnp.float32),
        ),
        mesh=mesh,
        scratch_types=dict(
            idx_all=pltpu.VMEM((2 * n_half, CHUNK), jnp.int32),
            buf0=pltpu.VMEM((CHUNK, d_half), jnp.float32),
            buf1=pltpu.VMEM((CHUNK, d_half), jnp.float32),
            acc=pltpu.VMEM_SHARED((n_rows_pad, d_half), jnp.float32),
            sem0=pltpu.SemaphoreType.DMA,
            sem1=pltpu.SemaphoreType.DMA,
            ssem0=pltpu.SemaphoreType.DMA,
            ssem1=pltpu.SemaphoreType.DMA,
        ),
    )
    def sc_kernel(x0, x1, edges2d, agg0, agg1,
                  idx_all, buf0, buf1, acc, sem0, sem1, ssem0, ssem1):
        cid = lax.axis_index("c")
        sid = lax.axis_index("s")
        zv = jnp.zeros((L,), jnp.float32)

        def fill_rows(i, _):
            def inner(j, _):
                buf0[i, pl.ds(j * L, L)] = zv
                return 0
            lax.fori_loop(0, d_half // L, inner, 0)
            return 0
        lax.fori_loop(0, CHUNK, fill_rows, 0)

        r0 = sid * rows_per_tile
        nfull = rows_per_tile // CHUNK
        rem = rows_per_tile - nfull * CHUNK
        for b in range(nfull):
            pltpu.sync_copy(buf0, acc.at[pl.ds(r0 + b * CHUNK, CHUNK)])
        if rem:
            pltpu.sync_copy(buf0.at[pl.ds(0, rem)],
                            acc.at[pl.ds(r0 + nfull * CHUNK, rem)])
        plsc.subcore_barrier()

        def gather(x_ref, k, buf, sem):
            pltpu.async_copy(x_ref.at[idx_all.at[2 * k]], buf, sem)

        def gwait(x_ref, buf, sem):
            pltpu.make_async_copy(x_ref.at[idx_all.at[0]], buf, sem).wait()

        def scat(k, buf, sem):
            pltpu.async_copy(buf, acc.at[idx_all.at[2 * k + 1]], sem,
                             add=True)

        def swait(buf, sem):
            pltpu.make_async_copy(buf, acc.at[idx_all.at[1]], sem).wait()

        def run_edges(x_ref):
            for h in range(2):
                erow = 2 * (sid * chunks_per_tile + h * n_half)
                pltpu.sync_copy(edges2d.at[pl.ds(erow, 2 * n_half)], idx_all)
                gather(x_ref, 0, buf0, sem0)
                gather(x_ref, 1, buf1, sem1)

                def body(g, _):
                    k0 = 2 * g
                    gwait(x_ref, buf0, sem0)
                    scat(k0, buf0, ssem0)
                    gwait(x_ref, buf1, sem1)
                    scat(k0 + 1, buf1, ssem1)
                    swait(buf0, ssem0)

                    @pl.when(g < n_half // 2 - 1)
                    def _():
                        gather(x_ref, k0 + 2, buf0, sem0)
                    swait(buf1, ssem1)

                    @pl.when(g < n_half // 2 - 1)
                    def _():
                        gather(x_ref, k0 + 3, buf1, sem1)
                    return 0
                lax.fori_loop(0, n_half // 2, body, 0)

        @pl.when(cid == 0)
        def _():
            run_edges(x0)

        @pl.when(cid == 1)
        def _():
            run_edges(x1)

        plsc.subcore_barrier()

        @pl.when(cid == 0)
        def _():
            pltpu.sync_copy(acc.at[pl.ds(r0, rows_per_tile)],
                            agg0.at[pl.ds(r0, rows_per_tile)])

        @pl.when(cid == 1)
        def _():
            pltpu.sync_copy(acc.at[pl.ds(r0, rows_per_tile)],
                            agg1.at[pl.ds(r0, rows_per_tile)])

    return sc_kernel


def _sc_count(n_rows_pad, n_chunk_rows):
    rows_per_tile = n_rows_pad // NS
    chunks_per_core = n_chunk_rows // NC
    chunks_per_tile = chunks_per_core // NS
    mesh = plsc.VectorSubcoreMesh(
        core_axis_name="c", subcore_axis_name="s", num_cores=NC, num_subcores=NS)

    @functools.partial(
        pl.kernel,
        out_type=jax.ShapeDtypeStruct((NC, n_rows_pad, CHUNK), jnp.float32),
        mesh=mesh,
        scratch_types=dict(
            idx_all=pltpu.VMEM((chunks_per_tile, CHUNK), jnp.int32),
            ones=pltpu.VMEM((CHUNK, CHUNK), jnp.float32),
            cnt=pltpu.VMEM_SHARED((n_rows_pad, CHUNK), jnp.float32),
            zero_part=pltpu.VMEM((CHUNK, CHUNK), jnp.float32),
            sem=pltpu.SemaphoreType.DMA,
        ),
    )
    def cnt_kernel(dstflat, cnt_out, idx_all, ones, cnt, zero_part, sem):
        cid = lax.axis_index("c")
        sid = lax.axis_index("s")
        zv = jnp.zeros((L,), jnp.float32)
        ov = jnp.full((L,), 1.0, jnp.float32)

        def fill(i, _):
            def inner(j, _):
                ones[i, pl.ds(j * L, L)] = ov
                zero_part[i, pl.ds(j * L, L)] = zv
                return 0
            lax.fori_loop(0, CHUNK // L, inner, 0)
            return 0
        lax.fori_loop(0, CHUNK, fill, 0)

        r0 = sid * rows_per_tile
        nfull = rows_per_tile // CHUNK
        rem = rows_per_tile - nfull * CHUNK
        for b in range(nfull):
            pltpu.sync_copy(zero_part, cnt.at[pl.ds(r0 + b * CHUNK, CHUNK)])
        if rem:
            pltpu.sync_copy(zero_part.at[pl.ds(0, rem)],
                            cnt.at[pl.ds(r0 + nfull * CHUNK, rem)])
        plsc.subcore_barrier()

        b0 = cid * chunks_per_core + sid * chunks_per_tile
        pltpu.sync_copy(dstflat.at[pl.ds(b0, chunks_per_tile)], idx_all)

        def body(k, _):
            pltpu.async_copy(ones, cnt.at[idx_all.at[k]], sem, add=True)
            return 0
        lax.fori_loop(0, chunks_per_tile, body, 0)

        def drain(k, _):
            pltpu.make_async_copy(ones, cnt.at[idx_all.at[0]], sem).wait()
            return 0
        lax.fori_loop(0, chunks_per_tile, drain, 0)

        plsc.subcore_barrier()

        @pl.when(cid == 0)
        def _():
            pltpu.sync_copy(cnt.at[pl.ds(r0, rows_per_tile)],
                            cnt_out.at[0, pl.ds(r0, rows_per_tile)])

        @pl.when(cid == 1)
        def _():
            pltpu.sync_copy(cnt.at[pl.ds(r0, rows_per_tile)],
                            cnt_out.at[1, pl.ds(r0, rows_per_tile)])

    return cnt_kernel


def _tc_tail(a0_ref, a1_ref, cnt_ref, w_ref, b_ref, tf_ref, out_ref):
    m = jnp.dot(a0_ref[...], w_ref[0:128, :],
                preferred_element_type=jnp.float32)
    m = m + jnp.dot(a1_ref[...], w_ref[128:256, :],
                    preferred_element_type=jnp.float32)
    deg = cnt_ref[0, :, 0:1] + cnt_ref[1, :, 0:1] + 1.0
    scale = lax.rsqrt(deg)
    left = jnp.maximum(m * scale + b_ref[...], 0.0)
    out_ref[...] = jnp.concatenate([left, jnp.abs(tf_ref[...])], axis=-1)


def kernel(x, inter_edge_index, W, bias, target_feat):
    n_src, d_in = x.shape
    n_tgt, tf_dim = target_feat.shape
    d_out = W.shape[1]
    d_half = d_in // 2
    e = inter_edge_index.shape[1]

    n_rows_pad = -(-(n_tgt + 1) // (NS * 8)) * (NS * 8)

    unit = NC * NS * CHUNK
    e_pad = -(-e // unit) * unit
    src = inter_edge_index[0]
    dst = inter_edge_index[1]
    if e_pad != e:
        pad = e_pad - e
        src = jnp.concatenate([src, jnp.zeros((pad,), jnp.int32)])
        dst = jnp.concatenate([dst, jnp.full((pad,), n_tgt, jnp.int32)])
    src2d = src.reshape(e_pad // CHUNK, CHUNK)
    dst2d = dst.reshape(e_pad // CHUNK, CHUNK)
    edges2d = jnp.stack([src2d, dst2d], axis=1).reshape(2 * (e_pad // CHUNK),
                                                        CHUNK)

    x0 = x[:, :d_half]
    x1 = x[:, d_half:]

    agg0, agg1 = _sc_aggregate(n_rows_pad, e_pad // CHUNK, d_half)(
        x0, x1, edges2d)
    cnt = _sc_count(n_rows_pad, e_pad // CHUNK)(dst2d)

    blk = 2000
    grid = n_tgt // blk
    out = pl.pallas_call(
        _tc_tail,
        grid=(grid,),
        in_specs=[
            pl.BlockSpec((blk, d_half), lambda i: (i, 0)),
            pl.BlockSpec((blk, d_half), lambda i: (i, 0)),
            pl.BlockSpec((NC, blk, CHUNK), lambda i: (0, i, 0)),
            pl.BlockSpec((d_in, d_out), lambda i: (0, 0)),
            pl.BlockSpec((1, d_out), lambda i: (0, 0)),
            pl.BlockSpec((blk, tf_dim), lambda i: (i, 0)),
        ],
        out_specs=pl.BlockSpec((blk, d_out + tf_dim), lambda i: (i, 0)),
        out_shape=jax.ShapeDtypeStruct((n_tgt, d_out + tf_dim), jnp.float32),
    )(agg0, agg1, cnt, W, bias.reshape(1, d_out), target_feat)
    return out


# speedup vs baseline: 1.2847x; 1.2847x over previous
import functools

import jax
import jax.numpy as jnp
from jax import lax
from jax.experimental import pallas as pl
from jax.experimental.pallas import tpu as pltpu
from jax.experimental.pallas import tpu_sc as plsc

NC = 2
NS = 16
L = 16

CHUNK = 128


def _sc_aggregate(n_rows_pad, n_chunk_rows, d_half):
    rows_per_tile = n_rows_pad // NS
    chunks_per_tile = n_chunk_rows // NS
    n_half = chunks_per_tile // 2
    mesh = plsc.VectorSubcoreMesh(
        core_axis_name="c", subcore_axis_name="s", num_cores=NC, num_subcores=NS)

    chunks_per_core = n_chunk_rows // NC
    cnt_chunks = chunks_per_core // NS

    @functools.partial(
        pl.kernel,
        out_type=(
            jax.ShapeDtypeStruct((n_rows_pad, d_half), jnp.float32),
            jax.ShapeDtypeStruct((n_rows_pad, d_half), jnp.float32),
            jax.ShapeDtypeStruct((NC, n_rows_pad, CHUNK), jnp.float32),
        ),
        mesh=mesh,
        scratch_types=dict(
            idx_all=pltpu.VMEM((2 * n_half, CHUNK), jnp.int32),
            buf0=pltpu.VMEM((CHUNK, d_half), jnp.float32),
            buf1=pltpu.VMEM((CHUNK, d_half), jnp.float32),
            acc=pltpu.VMEM_SHARED((n_rows_pad, d_half), jnp.float32),
            sem0=pltpu.SemaphoreType.DMA,
            sem1=pltpu.SemaphoreType.DMA,
        ),
    )
    def sc_kernel(x0, x1, edges2d, dstflat, agg0, agg1, cnt_out,
                  idx_all, buf0, buf1, acc, sem0, sem1):
        cid = lax.axis_index("c")
        sid = lax.axis_index("s")
        zv = jnp.zeros((L,), jnp.float32)
        ov = jnp.full((L,), 1.0, jnp.float32)

        def fill_buf(buf, val):
            def fr(i, _):
                def inner(j, _):
                    buf[i, pl.ds(j * L, L)] = val
                    return 0
                lax.fori_loop(0, d_half // L, inner, 0)
                return 0
            lax.fori_loop(0, CHUNK, fr, 0)

        fill_buf(buf0, zv)

        r0 = sid * rows_per_tile
        nfull = rows_per_tile // CHUNK
        rem = rows_per_tile - nfull * CHUNK
        for b in range(nfull):
            pltpu.sync_copy(buf0, acc.at[pl.ds(r0 + b * CHUNK, CHUNK)])
        if rem:
            pltpu.sync_copy(buf0.at[pl.ds(0, rem)],
                            acc.at[pl.ds(r0 + nfull * CHUNK, rem)])
        plsc.subcore_barrier()

        H = CHUNK // 2

        def gather(x_ref, k, buf, sem):
            pltpu.async_copy(x_ref.at[idx_all.at[2 * k, pl.ds(0, H)]],
                             buf.at[pl.ds(0, H)], sem)
            pltpu.async_copy(x_ref.at[idx_all.at[2 * k, pl.ds(H, H)]],
                             buf.at[pl.ds(H, H)], sem)

        def gwait(x_ref, buf, sem):
            pltpu.make_async_copy(x_ref.at[idx_all.at[0, pl.ds(0, H)]],
                                  buf.at[pl.ds(0, H)], sem).wait()
            pltpu.make_async_copy(x_ref.at[idx_all.at[0, pl.ds(0, H)]],
                                  buf.at[pl.ds(H, H)], sem).wait()

        def run_edges(x_ref):
            for h in range(2):
                erow = 2 * (sid * chunks_per_tile + h * n_half)
                pltpu.sync_copy(edges2d.at[pl.ds(erow, 2 * n_half)], idx_all)
                gather(x_ref, 0, buf0, sem0)
                gather(x_ref, 1, buf1, sem1)

                def body(g, _):
                    k0 = 2 * g
                    gwait(x_ref, buf0, sem0)
                    pltpu.sync_copy(buf0, acc.at[idx_all.at[2 * k0 + 1]],
                                    add=True)

                    @pl.when(g < n_half // 2 - 1)
                    def _():
                        gather(x_ref, k0 + 2, buf0, sem0)
                    gwait(x_ref, buf1, sem1)
                    pltpu.sync_copy(buf1, acc.at[idx_all.at[2 * k0 + 3]],
                                    add=True)

                    @pl.when(g < n_half // 2 - 1)
                    def _():
                        gather(x_ref, k0 + 3, buf1, sem1)
                    return 0
                lax.fori_loop(0, n_half // 2, body, 0)

        @pl.when(cid == 0)
        def _():
            run_edges(x0)

        @pl.when(cid == 1)
        def _():
            run_edges(x1)

        plsc.subcore_barrier()

        @pl.when(cid == 0)
        def _():
            pltpu.sync_copy(acc.at[pl.ds(r0, rows_per_tile)],
                            agg0.at[pl.ds(r0, rows_per_tile)])

        @pl.when(cid == 1)
        def _():
            pltpu.sync_copy(acc.at[pl.ds(r0, rows_per_tile)],
                            agg1.at[pl.ds(r0, rows_per_tile)])

        plsc.subcore_barrier()
        fill_buf(buf1, zv)
        fill_buf(buf0, ov)
        for b in range(nfull):
            pltpu.sync_copy(buf1, acc.at[pl.ds(r0 + b * CHUNK, CHUNK)])
        if rem:
            pltpu.sync_copy(buf1.at[pl.ds(0, rem)],
                            acc.at[pl.ds(r0 + nfull * CHUNK, rem)])
        plsc.subcore_barrier()

        b0 = cid * chunks_per_core + sid * cnt_chunks
        pltpu.sync_copy(dstflat.at[pl.ds(b0, cnt_chunks)],
                        idx_all.at[pl.ds(0, cnt_chunks)])

        def cbody(k, _):
            pltpu.async_copy(buf0, acc.at[idx_all.at[k]], sem0, add=True)
            return 0
        lax.fori_loop(0, cnt_chunks, cbody, 0)

        def cdrain(k, _):
            pltpu.make_async_copy(buf0, acc.at[idx_all.at[0]], sem0).wait()
            return 0
        lax.fori_loop(0, cnt_chunks, cdrain, 0)

        plsc.subcore_barrier()

        @pl.when(cid == 0)
        def _():
            pltpu.sync_copy(acc.at[pl.ds(r0, rows_per_tile)],
                            cnt_out.at[0, pl.ds(r0, rows_per_tile)])

        @pl.when(cid == 1)
        def _():
            pltpu.sync_copy(acc.at[pl.ds(r0, rows_per_tile)],
                            cnt_out.at[1, pl.ds(r0, rows_per_tile)])

    return sc_kernel


def _tc_tail(a0_ref, a1_ref, cnt_ref, w_ref, b_ref, tf_ref, out_ref):
    m = jnp.dot(a0_ref[...], w_ref[0:128, :],
                preferred_element_type=jnp.float32)
    m = m + jnp.dot(a1_ref[...], w_ref[128:256, :],
                    preferred_element_type=jnp.float32)
    deg = cnt_ref[0, :, 0:1] + cnt_ref[1, :, 0:1] + 1.0
    scale = lax.rsqrt(deg)
    left = jnp.maximum(m * scale + b_ref[...], 0.0)
    out_ref[...] = jnp.concatenate([left, jnp.abs(tf_ref[...])], axis=-1)


def kernel(x, inter_edge_index, W, bias, target_feat):
    n_src, d_in = x.shape
    n_tgt, tf_dim = target_feat.shape
    d_out = W.shape[1]
    d_half = d_in // 2
    e = inter_edge_index.shape[1]

    n_rows_pad = -(-(n_tgt + 1) // (NS * 8)) * (NS * 8)

    unit = NC * NS * CHUNK
    e_pad = -(-e // unit) * unit
    src = inter_edge_index[0]
    dst = inter_edge_index[1]
    if e_pad != e:
        pad = e_pad - e
        src = jnp.concatenate([src, jnp.zeros((pad,), jnp.int32)])
        dst = jnp.concatenate([dst, jnp.full((pad,), n_tgt, jnp.int32)])
    src2d = src.reshape(e_pad // CHUNK, CHUNK)
    dst2d = dst.reshape(e_pad // CHUNK, CHUNK)
    edges2d = jnp.stack([src2d, dst2d], axis=1).reshape(2 * (e_pad // CHUNK),
                                                        CHUNK)

    x0 = x[:, :d_half]
    x1 = x[:, d_half:]

    agg0, agg1, cnt = _sc_aggregate(n_rows_pad, e_pad // CHUNK, d_half)(
        x0, x1, edges2d, dst2d)

    blk = 2000
    grid = n_tgt // blk
    out = pl.pallas_call(
        _tc_tail,
        grid=(grid,),
        in_specs=[
            pl.BlockSpec((blk, d_half), lambda i: (i, 0)),
            pl.BlockSpec((blk, d_half), lambda i: (i, 0)),
            pl.BlockSpec((NC, blk, CHUNK), lambda i: (0, i, 0)),
            pl.BlockSpec((d_in, d_out), lambda i: (0, 0)),
            pl.BlockSpec((1, d_out), lambda i: (0, 0)),
            pl.BlockSpec((blk, tf_dim), lambda i: (i, 0)),
        ],
        out_specs=pl.BlockSpec((blk, d_out + tf_dim), lambda i: (i, 0)),
        out_shape=jax.ShapeDtypeStruct((n_tgt, d_out + tf_dim), jnp.float32),
    )(agg0, agg1, cnt, W, bias.reshape(1, d_out), target_feat)
    return out
